# baseline (speedup 1.0000x reference)
import functools

import jax
import jax.numpy as jnp
from jax import lax
from jax.experimental import pallas as pl
from jax.experimental.pallas import tpu as pltpu
from jax.experimental.pallas import tpu_sc as plsc

B, L, V, D = 1024, 200, 1000000, 64
NROWS = B * L
LH = L // 2
NROWS_H = LH * B
NW = 32
RW = NROWS_H // NW
CHUNK = 320
SUB = 64
NSUB = CHUNK // SUB
NCHUNK = RW // CHUNK

H = 524288
NPAIR = 8192
NREP = H // NPAIR


def _tc_repack(table_t, eye64):
    def body(xl_ref, xr_ref, eye_ref, o_ref):
        e = eye_ref[...]
        tl = lax.dot_general(
            xl_ref[...], e, (((0,), (0,)), ((), ())),
            preferred_element_type=jnp.float32)
        tr = lax.dot_general(
            xr_ref[...], e, (((0,), (0,)), ((), ())),
            preferred_element_type=jnp.float32)
        o_ref[:, :D] = tl
        o_ref[:, D:] = tr

    return pl.pallas_call(
        body,
        grid=(NREP,),
        in_specs=[
            pl.BlockSpec((D, NPAIR), lambda i: (0, i)),
            pl.BlockSpec((D, NPAIR), lambda i: (0, jnp.minimum(i + NREP, V // NPAIR))),
            pl.BlockSpec((D, D), lambda i: (0, 0)),
        ],
        out_specs=pl.BlockSpec((NPAIR, 128), lambda i: (i, 0)),
        out_shape=jax.ShapeDtypeStruct((H, 128), jnp.float32),
    )(table_t, table_t, eye64)


def _sc_gather_pairs(idx2, table2):
    mesh = plsc.VectorSubcoreMesh(core_axis_name="c", subcore_axis_name="s")

    @functools.partial(
        pl.kernel,
        out_type=jax.ShapeDtypeStruct((NROWS_H, 128), jnp.float32),
        mesh=mesh,
        scratch_types=[
            pltpu.VMEM((RW,), jnp.int32),
            pltpu.VMEM((2, CHUNK, 128), jnp.float32),
            pltpu.SemaphoreType.DMA,
            pltpu.SemaphoreType.DMA,
            pltpu.SemaphoreType.DMA,
        ],
    )
    def gather_kernel(idx_hbm, table_hbm, out_hbm, idx_v, rows_v,
                      gsem0, gsem1, ssem):
        wid = lax.axis_index("s") * 2 + lax.axis_index("c")
        base = wid * RW
        pltpu.sync_copy(idx_hbm.at[pl.ds(base, RW)], idx_v)
        gsems = (gsem0, gsem1)

        def fire(slot, g, sem):
            cps = []
            for j in range(NSUB):
                cps.append(
                    pltpu.async_copy(
                        table_hbm.at[idx_v.at[pl.ds(g * CHUNK + j * SUB, SUB)]],
                        rows_v.at[slot].at[pl.ds(j * SUB, SUB)],
                        sem,
                    )
                )
            return cps

        def drain(cps):
            for cp in cps:
                cp.wait()

        cps0 = fire(0, 0, gsem0)

        def pair_body(h, carry):
            a = 2 * h
            b = a + 1
            for j in range(NSUB):
                pltpu.make_async_copy(
                    table_hbm.at[idx_v.at[pl.ds(a * CHUNK + j * SUB, SUB)]],
                    rows_v.at[0].at[pl.ds(j * SUB, SUB)],
                    gsem0,
                ).wait()
            cb = fire(1, b, gsem1)
            st_a = pltpu.async_copy(
                rows_v.at[0], out_hbm.at[pl.ds(base + a * CHUNK, CHUNK)], ssem)
            drain(cb)
            st_a.wait()

            @pl.when(h + 1 < NCHUNK // 2)
            def _():
                fire(0, a + 2, gsem0)

            st_b = pltpu.async_copy(
                rows_v.at[1], out_hbm.at[pl.ds(base + b * CHUNK, CHUNK)], ssem)
            st_b.wait()
            return carry

        lax.fori_loop(0, NCHUNK // 2, pair_body, 0)

    return gather_kernel(idx2, table2)


LB = 10


def _tc_out1t(g_half, par3, ph_t, pos_r, unit_embed, eyelr, off, prev=None):
    def body(g_ref, par_ref, ph_ref, pos_ref, ue_ref, eye_ref, out_ref):
        g3 = g_ref[...].reshape(LB, B, 128)
        pm = par_ref[...]
        ue = ue_ref[...]
        el = eye_ref[0]
        er = eye_ref[1]
        for j in range(LB):
            gl = lax.dot_general(
                el, g3[j], (((1,), (1,)), ((), ())),
                preferred_element_type=jnp.float32)
            gr = lax.dot_general(
                er, g3[j], (((1,), (1,)), ((), ())),
                preferred_element_type=jnp.float32)
            half = gl + pm[j] * (gr - gl)
            unit = lax.dot_general(
                ue, ph_ref[:, 0, j, :],
                (((0,), (0,)), ((), ())),
                preferred_element_type=jnp.float32,
            )
            out_ref[j] = half + unit + pos_ref[0, :, j][:, None]

    in_specs = [
        pl.BlockSpec((LB * B, 128), lambda i: (i, 0)),
        pl.BlockSpec((LB, 1, B), lambda i: (i + off, 0, 0)),
        pl.BlockSpec((7, 1, LB, B), lambda i: (0, i + off, 0, 0)),
        pl.BlockSpec((1, D, LB), lambda i: (i + off, 0, 0)),
        pl.BlockSpec((7, D), lambda i: (0, 0)),
        pl.BlockSpec((2, D, 128), lambda i: (0, 0, 0)),
    ]
    args = [g_half, par3, ph_t, pos_r, unit_embed, eyelr]
    aliases = {}
    if prev is not None:
        def wrapped(prev_ref, *refs):
            body(*refs)
        fn = wrapped
        in_specs = [pl.BlockSpec(memory_space=pl.ANY)] + in_specs
        args = [prev] + args
        aliases = {0: 0}
    else:
        fn = body
    return pl.pallas_call(
        fn,
        grid=(LH // LB,),
        in_specs=in_specs,
        out_specs=pl.BlockSpec((LB, D, B), lambda i: (i + off, 0, 0)),
        out_shape=jax.ShapeDtypeStruct((L, D, B), jnp.float32),
        input_output_aliases=aliases,
    )(*args)


LB2 = 8


def _tc_out2t(meta_t, pad_t, case_embed):
    def body(meta_ref, pad_ref, case_ref, out_ref):
        mt = meta_ref[...]
        case = case_ref[...]
        for j in range(LB2):
            rhs = mt * pad_ref[0, :, j][:, None]
            out_ref[j] = lax.dot_general(
                case, rhs,
                (((0,), (0,)), ((), ())),
                preferred_element_type=jnp.float32,
            )

    return pl.pallas_call(
        body,
        grid=(L // LB2,),
        in_specs=[
            pl.BlockSpec((D, B), lambda i: (0, 0)),
            pl.BlockSpec((1, D, LB2), lambda i: (i, 0, 0)),
            pl.BlockSpec((D, D), lambda i: (0, 0)),
        ],
        out_specs=pl.BlockSpec((LB2, D, B), lambda i: (i, 0, 0)),
        out_shape=jax.ShapeDtypeStruct((L, D, B), jnp.float32),
    )(meta_t, pad_t, case_embed)


def kernel(sequence, meta_info, ph_dimensions, token_table, pos_table,
           case_embed, unit_embed, padding):
    table_t = jnp.transpose(token_table, (1, 0))
    eye64 = jnp.eye(D, dtype=jnp.float32)
    table2 = _tc_repack(table_t, eye64)

    seq_t = jnp.transpose(sequence, (1, 0))
    idx_t = seq_t.reshape(NROWS).astype(jnp.int32)
    high = idx_t >= H
    idx2 = jnp.where(high, idx_t - H, idx_t)
    par3 = high.astype(jnp.float32).reshape(L, 1, B)
    eyelr = jnp.stack([jnp.eye(D, 128, dtype=jnp.float32),
                       jnp.eye(D, 128, k=D, dtype=jnp.float32)])

    g_a = _sc_gather_pairs(idx2[:NROWS_H], table2)
    g_b = _sc_gather_pairs(idx2[NROWS_H:], table2)

    ph_t = jnp.transpose(ph_dimensions.astype(jnp.float32), (2, 1, 0))
    ph_t = ph_t.reshape(7, L // LB, LB, B)
    pos_t = jnp.transpose(pos_table, (1, 0))
    pos_r = jnp.transpose(pos_t.reshape(D, L // LB, LB), (1, 0, 2))
    out1t = _tc_out1t(g_a, par3, ph_t, pos_r, unit_embed, eyelr, 0)
    out1t = _tc_out1t(g_b, par3, ph_t, pos_r, unit_embed, eyelr,
                      LH // LB, prev=out1t)

    meta_t = jnp.transpose(meta_info, (1, 0))
    pad_t = jnp.transpose(padding, (1, 0))
    pad_r = jnp.transpose(pad_t.reshape(D, L // LB2, LB2), (1, 0, 2))
    out2t = _tc_out2t(meta_t, pad_r, case_embed)

    out1 = jnp.transpose(out1t, (2, 0, 1))
    out2 = jnp.transpose(out2t, (2, 0, 1))
    return (out1, out2)


# speedup vs baseline: 1.0109x; 1.0109x over previous
import functools

import jax
import jax.numpy as jnp
from jax import lax
from jax.experimental import pallas as pl
from jax.experimental.pallas import tpu as pltpu
from jax.experimental.pallas import tpu_sc as plsc

B, L, V, D = 1024, 200, 1000000, 64
NROWS = B * L
NW = 32
RW = NROWS // NW
CHUNK = 320
SUB = 64
NSUB = CHUNK // SUB
NCHUNK = RW // CHUNK

H = 524288
NPAIR = 8192
NREP = H // NPAIR


def _tc_repack(table_t, eye64):
    def body(xl_ref, xr_ref, eye_ref, o_ref):
        del eye_ref
        o_ref[:, :D] = jnp.transpose(xl_ref[...], (1, 0))
        o_ref[:, D:] = jnp.transpose(xr_ref[...], (1, 0))

    return pl.pallas_call(
        body,
        grid=(NREP,),
        in_specs=[
            pl.BlockSpec((D, NPAIR), lambda i: (0, i)),
            pl.BlockSpec((D, NPAIR), lambda i: (0, jnp.minimum(i + NREP, V // NPAIR))),
            pl.BlockSpec((D, D), lambda i: (0, 0)),
        ],
        out_specs=pl.BlockSpec((NPAIR, 128), lambda i: (i, 0)),
        out_shape=jax.ShapeDtypeStruct((H, 128), jnp.float32),
    )(table_t, table_t, eye64)


def _sc_gather_pairs(idx2, table2):
    mesh = plsc.VectorSubcoreMesh(core_axis_name="c", subcore_axis_name="s")

    @functools.partial(
        pl.kernel,
        out_type=jax.ShapeDtypeStruct((NROWS, 128), jnp.float32),
        mesh=mesh,
        scratch_types=[
            pltpu.VMEM((RW,), jnp.int32),
            pltpu.VMEM((2, CHUNK, 128), jnp.float32),
            pltpu.SemaphoreType.DMA,
            pltpu.SemaphoreType.DMA,
            pltpu.SemaphoreType.DMA,
        ],
    )
    def gather_kernel(idx_hbm, table_hbm, out_hbm, idx_v, rows_v,
                      gsem0, gsem1, ssem):
        wid = lax.axis_index("s") * 2 + lax.axis_index("c")
        base = wid * RW
        pltpu.sync_copy(idx_hbm.at[pl.ds(base, RW)], idx_v)
        gsems = (gsem0, gsem1)

        def fire(slot, g, sem):
            cps = []
            for j in range(NSUB):
                cps.append(
                    pltpu.async_copy(
                        table_hbm.at[idx_v.at[pl.ds(g * CHUNK + j * SUB, SUB)]],
                        rows_v.at[slot].at[pl.ds(j * SUB, SUB)],
                        sem,
                    )
                )
            return cps

        def drain(cps):
            for cp in cps:
                cp.wait()

        cps0 = fire(0, 0, gsem0)

        def pair_body(h, carry):
            a = 2 * h
            b = a + 1
            for j in range(NSUB):
                pltpu.make_async_copy(
                    table_hbm.at[idx_v.at[pl.ds(a * CHUNK + j * SUB, SUB)]],
                    rows_v.at[0].at[pl.ds(j * SUB, SUB)],
                    gsem0,
                ).wait()
            cb = fire(1, b, gsem1)
            st_a = pltpu.async_copy(
                rows_v.at[0], out_hbm.at[pl.ds(base + a * CHUNK, CHUNK)], ssem)
            drain(cb)
            st_a.wait()

            @pl.when(h + 1 < NCHUNK // 2)
            def _():
                fire(0, a + 2, gsem0)

            st_b = pltpu.async_copy(
                rows_v.at[1], out_hbm.at[pl.ds(base + b * CHUNK, CHUNK)], ssem)
            st_b.wait()
            return carry

        lax.fori_loop(0, NCHUNK // 2, pair_body, 0)

    return gather_kernel(idx2, table2)


LB = 8


def _tc_out1t(g128, par3, ph_t, pos_t, unit_embed, eyelr):
    def body(g_ref, par_ref, ph_ref, pos_ref, ue_ref, eye_ref, out_ref):
        g3 = g_ref[...].reshape(LB, B, 128)
        pm = par_ref[...]
        ue = ue_ref[...]
        el = eye_ref[0]
        er = eye_ref[1]
        for j in range(LB):
            gl = lax.dot_general(
                el, g3[j], (((1,), (1,)), ((), ())),
                preferred_element_type=jnp.float32)
            gr = lax.dot_general(
                er, g3[j], (((1,), (1,)), ((), ())),
                preferred_element_type=jnp.float32)
            half = gl + pm[j] * (gr - gl)
            unit = lax.dot_general(
                ue, ph_ref[:, j, :],
                (((0,), (0,)), ((), ())),
                preferred_element_type=jnp.float32,
            )
            out_ref[j] = half + unit + pos_ref[0, :, j][:, None]

    return pl.pallas_call(
        body,
        grid=(L // LB,),
        in_specs=[
            pl.BlockSpec((LB * B, 128), lambda i: (i, 0)),
            pl.BlockSpec((LB, 1, B), lambda i: (i, 0, 0)),
            pl.BlockSpec((7, LB, B), lambda i: (0, i, 0)),
            pl.BlockSpec((1, D, LB), lambda i: (i, 0, 0)),
            pl.BlockSpec((7, D), lambda i: (0, 0)),
            pl.BlockSpec((2, D, 128), lambda i: (0, 0, 0)),
        ],
        out_specs=pl.BlockSpec((LB, D, B), lambda i: (i, 0, 0)),
        out_shape=jax.ShapeDtypeStruct((L, D, B), jnp.float32),
    )(g128, par3, ph_t, pos_t, unit_embed, eyelr)


LB2 = 8


def _tc_out2t(meta_t, pad_t, case_embed):
    def body(meta_ref, pad_ref, case_ref, out_ref):
        mt = meta_ref[...]
        case = case_ref[...]
        for j in range(LB2):
            rhs = mt * pad_ref[0, :, j][:, None]
            out_ref[j] = lax.dot_general(
                case, rhs,
                (((0,), (0,)), ((), ())),
                preferred_element_type=jnp.float32,
            )

    return pl.pallas_call(
        body,
        grid=(L // LB2,),
        in_specs=[
            pl.BlockSpec((D, B), lambda i: (0, 0)),
            pl.BlockSpec((1, D, LB2), lambda i: (i, 0, 0)),
            pl.BlockSpec((D, D), lambda i: (0, 0)),
        ],
        out_specs=pl.BlockSpec((LB2, D, B), lambda i: (i, 0, 0)),
        out_shape=jax.ShapeDtypeStruct((L, D, B), jnp.float32),
    )(meta_t, pad_t, case_embed)


def kernel(sequence, meta_info, ph_dimensions, token_table, pos_table,
           case_embed, unit_embed, padding):
    table_t = jnp.transpose(token_table, (1, 0))
    eye64 = jnp.eye(D, dtype=jnp.float32)
    table2 = _tc_repack(table_t, eye64)

    seq_t = jnp.transpose(sequence, (1, 0))
    idx_t = seq_t.reshape(NROWS).astype(jnp.int32)
    high = idx_t >= H
    idx2 = jnp.where(high, idx_t - H, idx_t)
    par3 = high.astype(jnp.float32).reshape(L, 1, B)
    eyelr = jnp.stack([jnp.eye(D, 128, dtype=jnp.float32),
                       jnp.eye(D, 128, k=D, dtype=jnp.float32)])

    g128 = _sc_gather_pairs(idx2, table2)

    ph_t = jnp.transpose(ph_dimensions.astype(jnp.float32), (2, 1, 0))
    pos_t = jnp.transpose(pos_table, (1, 0))
    pos_r = jnp.transpose(pos_t.reshape(D, L // LB, LB), (1, 0, 2))
    out1t = _tc_out1t(g128, par3, ph_t, pos_r, unit_embed, eyelr)

    meta_t = jnp.transpose(meta_info, (1, 0))
    pad_t = jnp.transpose(padding, (1, 0))
    pad_r = jnp.transpose(pad_t.reshape(D, L // LB2, LB2), (1, 0, 2))
    out2t = _tc_out2t(meta_t, pad_r, case_embed)

    out1 = jnp.transpose(out1t, (2, 0, 1))
    out2 = jnp.transpose(out2t, (2, 0, 1))
    return (out1, out2)


# speedup vs baseline: 1.1837x; 1.1709x over previous
import functools

import jax
import jax.numpy as jnp
from jax import lax
from jax.experimental import pallas as pl
from jax.experimental.pallas import tpu as pltpu
from jax.experimental.pallas import tpu_sc as plsc

B, L, V, D = 1024, 200, 1000000, 64
NROWS = B * L
NW = 32
RW = NROWS // NW
CHUNK = 320
SUB = 64
NSUB = CHUNK // SUB
NCHUNK = RW // CHUNK

H = 524288
NPAIR = 8192
NREP = H // NPAIR


def _tc_repack(table_t, eye64):
    def body(xl_ref, xr_ref, eye_ref, o_ref):
        del eye_ref
        xcat = jnp.concatenate([xl_ref[...], xr_ref[...]], axis=0)
        o_ref[...] = jnp.transpose(xcat, (1, 0))

    return pl.pallas_call(
        body,
        grid=(NREP,),
        in_specs=[
            pl.BlockSpec((D, NPAIR), lambda i: (0, i)),
            pl.BlockSpec((D, NPAIR), lambda i: (0, jnp.minimum(i + NREP, V // NPAIR))),
            pl.BlockSpec((D, D), lambda i: (0, 0)),
        ],
        out_specs=pl.BlockSpec((NPAIR, 128), lambda i: (i, 0)),
        out_shape=jax.ShapeDtypeStruct((H, 128), jnp.float32),
    )(table_t, table_t, eye64)


def _sc_gather_pairs(idx2, table2):
    mesh = plsc.VectorSubcoreMesh(core_axis_name="c", subcore_axis_name="s")

    @functools.partial(
        pl.kernel,
        out_type=jax.ShapeDtypeStruct((NROWS, 128), jnp.float32),
        mesh=mesh,
        scratch_types=[
            pltpu.VMEM((RW,), jnp.int32),
            pltpu.VMEM((2, CHUNK, 128), jnp.float32),
            pltpu.SemaphoreType.DMA,
            pltpu.SemaphoreType.DMA,
            pltpu.SemaphoreType.DMA,
        ],
    )
    def gather_kernel(idx_hbm, table_hbm, out_hbm, idx_v, rows_v,
                      gsem0, gsem1, ssem):
        wid = lax.axis_index("s") * 2 + lax.axis_index("c")
        base = wid * RW
        pltpu.sync_copy(idx_hbm.at[pl.ds(base, RW)], idx_v)
        gsems = (gsem0, gsem1)

        def fire(slot, g, sem):
            cps = []
            for j in range(NSUB):
                cps.append(
                    pltpu.async_copy(
                        table_hbm.at[idx_v.at[pl.ds(g * CHUNK + j * SUB, SUB)]],
                        rows_v.at[slot].at[pl.ds(j * SUB, SUB)],
                        sem,
                    )
                )
            return cps

        def drain(cps):
            for cp in cps:
                cp.wait()

        cps0 = fire(0, 0, gsem0)

        def pair_body(h, carry):
            a = 2 * h
            b = a + 1
            for j in range(NSUB):
                pltpu.make_async_copy(
                    table_hbm.at[idx_v.at[pl.ds(a * CHUNK + j * SUB, SUB)]],
                    rows_v.at[0].at[pl.ds(j * SUB, SUB)],
                    gsem0,
                ).wait()
            cb = fire(1, b, gsem1)
            st_a = pltpu.async_copy(
                rows_v.at[0], out_hbm.at[pl.ds(base + a * CHUNK, CHUNK)], ssem)
            drain(cb)
            st_a.wait()

            @pl.when(h + 1 < NCHUNK // 2)
            def _():
                fire(0, a + 2, gsem0)

            st_b = pltpu.async_copy(
                rows_v.at[1], out_hbm.at[pl.ds(base + b * CHUNK, CHUNK)], ssem)
            st_b.wait()
            return carry

        lax.fori_loop(0, NCHUNK // 2, pair_body, 0)

    return gather_kernel(idx2, table2)


LB = 8


def _tc_out1t(g128, par3, ph_t, pos_t, unit_embed, eyelr):
    def body(g_ref, par_ref, ph_ref, pos_ref, ue_ref, eye_ref, out_ref):
        g3 = g_ref[...].reshape(LB, B, 128)
        pm = par_ref[...]
        ue = ue_ref[...]
        el = eye_ref[0]
        er = eye_ref[1]
        for j in range(LB):
            gl = lax.dot_general(
                el, g3[j], (((1,), (1,)), ((), ())),
                preferred_element_type=jnp.float32)
            gr = lax.dot_general(
                er, g3[j], (((1,), (1,)), ((), ())),
                preferred_element_type=jnp.float32)
            half = gl + pm[j] * (gr - gl)
            unit = lax.dot_general(
                ue, ph_ref[:, j, :],
                (((0,), (0,)), ((), ())),
                preferred_element_type=jnp.float32,
            )
            out_ref[j] = half + unit + pos_ref[0, :, j][:, None]

    return pl.pallas_call(
        body,
        grid=(L // LB,),
        in_specs=[
            pl.BlockSpec((LB * B, 128), lambda i: (i, 0)),
            pl.BlockSpec((LB, 1, B), lambda i: (i, 0, 0)),
            pl.BlockSpec((7, LB, B), lambda i: (0, i, 0)),
            pl.BlockSpec((1, D, LB), lambda i: (i, 0, 0)),
            pl.BlockSpec((7, D), lambda i: (0, 0)),
            pl.BlockSpec((2, D, 128), lambda i: (0, 0, 0)),
        ],
        out_specs=pl.BlockSpec((LB, D, B), lambda i: (i, 0, 0)),
        out_shape=jax.ShapeDtypeStruct((L, D, B), jnp.float32),
    )(g128, par3, ph_t, pos_t, unit_embed, eyelr)


LB2 = 8


def _tc_out2t(meta_t, pad_t, case_embed):
    def body(meta_ref, pad_ref, case_ref, out_ref):
        mt = meta_ref[...]
        case = case_ref[...]
        for j in range(LB2):
            rhs = mt * pad_ref[0, :, j][:, None]
            out_ref[j] = lax.dot_general(
                case, rhs,
                (((0,), (0,)), ((), ())),
                preferred_element_type=jnp.float32,
            )

    return pl.pallas_call(
        body,
        grid=(L // LB2,),
        in_specs=[
            pl.BlockSpec((D, B), lambda i: (0, 0)),
            pl.BlockSpec((1, D, LB2), lambda i: (i, 0, 0)),
            pl.BlockSpec((D, D), lambda i: (0, 0)),
        ],
        out_specs=pl.BlockSpec((LB2, D, B), lambda i: (i, 0, 0)),
        out_shape=jax.ShapeDtypeStruct((L, D, B), jnp.float32),
    )(meta_t, pad_t, case_embed)


def kernel(sequence, meta_info, ph_dimensions, token_table, pos_table,
           case_embed, unit_embed, padding):
    table_t = jnp.transpose(token_table, (1, 0))
    eye64 = jnp.eye(D, dtype=jnp.float32)
    table2 = _tc_repack(table_t, eye64)

    seq_t = jnp.transpose(sequence, (1, 0))
    idx_t = seq_t.reshape(NROWS).astype(jnp.int32)
    high = idx_t >= H
    idx2 = jnp.where(high, idx_t - H, idx_t)
    par3 = high.astype(jnp.float32).reshape(L, 1, B)
    eyelr = jnp.stack([jnp.eye(D, 128, dtype=jnp.float32),
                       jnp.eye(D, 128, k=D, dtype=jnp.float32)])

    g128 = _sc_gather_pairs(idx2, table2)

    ph_t = jnp.transpose(ph_dimensions.astype(jnp.float32), (2, 1, 0))
    pos_t = jnp.transpose(pos_table, (1, 0))
    pos_r = jnp.transpose(pos_t.reshape(D, L // LB, LB), (1, 0, 2))
    out1t = _tc_out1t(g128, par3, ph_t, pos_r, unit_embed, eyelr)

    meta_t = jnp.transpose(meta_info, (1, 0))
    pad_t = jnp.transpose(padding, (1, 0))
    pad_r = jnp.transpose(pad_t.reshape(D, L // LB2, LB2), (1, 0, 2))
    out2t = _tc_out2t(meta_t, pad_r, case_embed)

    out1 = jnp.transpose(out1t, (2, 0, 1))
    out2 = jnp.transpose(out2t, (2, 0, 1))
    return (out1, out2)
